# baseline (speedup 1.0000x reference)
import functools

import jax
import jax.numpy as jnp
from jax import lax
from jax.experimental import pallas as pl
from jax.experimental.pallas import tpu as pltpu
from jax.experimental.pallas import tpu_sc as plsc

D_EMBED = 64
SCALE = float(D_EMBED ** 0.5)

B_TOTAL = 4096 * 200
NW = 32
B_PER_W = B_TOTAL // NW
CHUNK = 256
N_CHUNKS = B_PER_W // CHUNK
W_PAIR = 2 * D_EMBED

_mesh = plsc.VectorSubcoreMesh(core_axis_name="c", subcore_axis_name="s")


@functools.partial(
    pl.kernel,
    mesh=_mesh,
    out_type=jax.ShapeDtypeStruct((B_TOTAL * D_EMBED,), jnp.float32),
    scratch_types=[
        pltpu.VMEM((B_PER_W,), jnp.int32),
        pltpu.VMEM((2 * CHUNK, W_PAIR), jnp.float32),
        pltpu.VMEM((2 * CHUNK * D_EMBED,), jnp.float32),
        pltpu.SemaphoreType.DMA,
        pltpu.SemaphoreType.DMA,
    ],
    compiler_params=pltpu.CompilerParams(use_tc_tiling_on_sc=True),
)
def _gather_scale(idx_hbm, table_hbm, out_hbm, idx_v, rows_v, stage_v,
                  sem0, sem1):
    sems = (sem0, sem1)
    wid = lax.axis_index("s") * 2 + lax.axis_index("c")
    base = wid * B_PER_W
    pltpu.sync_copy(idx_hbm.at[pl.ds(base, B_PER_W)], idx_v)

    def fire(c, b):
        for i in range(CHUNK // 16):
            idx16 = idx_v[pl.ds(c * CHUNK + i * 16, 16)]
            pair16 = lax.shift_right_logical(idx16, 1)
            pltpu.async_copy(
                table_hbm.at[pair16],
                rows_v.at[pl.ds(b * CHUNK + i * 16, 16)], sems[b])

    def process(c, b):
        for i in range(CHUNK // 16):
            idx16 = idx_v[pl.ds(c * CHUNK + i * 16, 16)]
            pair16 = lax.shift_right_logical(idx16, 1)
            pltpu.make_async_copy(
                table_hbm.at[pair16],
                rows_v.at[pl.ds(b * CHUNK + i * 16, 16)], sems[b]
            ).wait()

        def select_body(g, carry):
            idx16 = idx_v[pl.ds(c * CHUNK + g * 16, 16)]
            offv = lax.shift_left(lax.bitwise_and(idx16, 1), 6)
            for r in range(16):
                off_r = offv[r]
                row = g * 16 + r
                for j in range(D_EMBED // 16):
                    v = rows_v[b * CHUNK + row, pl.ds(off_r + j * 16, 16)]
                    stage_v[pl.ds((b * CHUNK + row) * D_EMBED + j * 16, 16)] = (
                        v * SCALE)
            return carry

        lax.fori_loop(0, CHUNK // 16, select_body, 0)
        pltpu.sync_copy(
            stage_v.at[pl.ds(b * CHUNK * D_EMBED, CHUNK * D_EMBED)],
            out_hbm.at[pl.ds((base + c * CHUNK) * D_EMBED, CHUNK * D_EMBED)])

    fire(0, 0)

    def pair_body(i, carry):
        c0 = 2 * i
        fire(c0 + 1, 1)
        process(c0, 0)
        fire(c0 + 2, 0)
        process(c0 + 1, 1)
        return carry

    lax.fori_loop(0, (N_CHUNKS - 2) // 2, pair_body, 0)
    fire(N_CHUNKS - 1, 1)
    process(N_CHUNKS - 2, 0)
    process(N_CHUNKS - 1, 1)


def kernel(inp, emb_weight):
    idx = inp.reshape(B_TOTAL)
    if idx.dtype != jnp.int32:
        idx = idx.astype(jnp.int32)
    table_pairs = emb_weight.reshape(emb_weight.shape[0] // 2, W_PAIR)
    out = _gather_scale(idx, table_pairs)
    return out.reshape(inp.shape[0], inp.shape[1], D_EMBED)


# speedup vs baseline: 1.3051x; 1.3051x over previous
import functools

import jax
import jax.numpy as jnp
from jax import lax
from jax.experimental import pallas as pl
from jax.experimental.pallas import tpu as pltpu
from jax.experimental.pallas import tpu_sc as plsc

D_EMBED = 64
SCALE = float(D_EMBED ** 0.5)

B_TOTAL = 4096 * 200
NW = 32
B_PER_W = B_TOTAL // NW
CHUNK = 512
N_CHUNKS = B_PER_W // CHUNK

_mesh = plsc.VectorSubcoreMesh(core_axis_name="c", subcore_axis_name="s")


@functools.partial(
    pl.kernel,
    mesh=_mesh,
    out_type=jax.ShapeDtypeStruct((B_TOTAL, D_EMBED), jnp.float32),
    scratch_types=[
        pltpu.VMEM((B_PER_W,), jnp.int32),
        pltpu.VMEM((2 * CHUNK, D_EMBED), jnp.float32),
        pltpu.SemaphoreType.DMA,
        pltpu.SemaphoreType.DMA,
        pltpu.SemaphoreType.DMA,
        pltpu.SemaphoreType.DMA,
    ],
    compiler_params=pltpu.CompilerParams(use_tc_tiling_on_sc=False),
)
def _gather_scale(idx_hbm, table_hbm, out_hbm, idx_v, rows_v,
                  gsem0, gsem1, ssem0, ssem1):
    gsems = (gsem0, gsem1)
    ssems = (ssem0, ssem1)
    wid = lax.axis_index("s") * 2 + lax.axis_index("c")
    base = wid * B_PER_W
    pltpu.sync_copy(idx_hbm.at[pl.ds(base, B_PER_W)], idx_v)

    def fire(c, b):
        pltpu.async_copy(
            table_hbm.at[idx_v.at[pl.ds(c * CHUNK, CHUNK)]],
            rows_v.at[pl.ds(b * CHUNK, CHUNK)], gsems[b])

    def process(c, b):
        pltpu.make_async_copy(
            table_hbm.at[idx_v.at[pl.ds(c * CHUNK, CHUNK)]],
            rows_v.at[pl.ds(b * CHUNK, CHUNK)], gsems[b]
        ).wait()

        def scale_body(rr, carry):
            for j in range(D_EMBED // 16):
                sl = (rr, pl.ds(j * 16, 16))
                rows_v[sl] = rows_v[sl] * SCALE
            return carry

        lax.fori_loop(b * CHUNK, (b + 1) * CHUNK, scale_body, 0)
        pltpu.async_copy(rows_v.at[pl.ds(b * CHUNK, CHUNK)],
                         out_hbm.at[pl.ds(base + c * CHUNK, CHUNK)], ssems[b])

    def wait_store(c, b):
        pltpu.make_async_copy(
            rows_v.at[pl.ds(b * CHUNK, CHUNK)],
            out_hbm.at[pl.ds(base + c * CHUNK, CHUNK)], ssems[b]
        ).wait()

    fire(0, 0)

    def pair_body(i, carry):
        c0 = 2 * i
        fire(c0 + 1, 1)
        process(c0, 0)
        wait_store(c0, 0)
        fire(c0 + 2, 0)
        process(c0 + 1, 1)
        wait_store(c0 + 1, 1)
        return carry

    lax.fori_loop(0, (N_CHUNKS - 2) // 2, pair_body, 0)
    fire(N_CHUNKS - 1, 1)
    process(N_CHUNKS - 2, 0)
    wait_store(N_CHUNKS - 2, 0)
    process(N_CHUNKS - 1, 1)
    wait_store(N_CHUNKS - 1, 1)


def kernel(inp, emb_weight):
    idx = inp.reshape(B_TOTAL)
    if idx.dtype != jnp.int32:
        idx = idx.astype(jnp.int32)
    out = _gather_scale(idx, emb_weight)
    return out.reshape(inp.shape[0], inp.shape[1], D_EMBED)
